# baseline (speedup 1.0000x reference)
import dataclasses
import functools

import jax
import jax.numpy as jnp
from jax import lax
from jax.experimental import pallas as pl
from jax.experimental.pallas import tpu as pltpu
from jax.experimental.pallas import tpu_sc as plsc

N = 10000
E = 320000
D = 128
NR = 10240
NC, NS = 2, 16
NT = NC * NS
K = 128
CH = 80
RING = 4
EPT = K * CH
EPAD = NT * EPT
RPS = NR // NS

_mesh = plsc.VectorSubcoreMesh(core_axis_name="c", subcore_axis_name="s")

_sc_params = pltpu.CompilerParams()
if "needs_layout_passes" in pltpu.CompilerParams.__dataclass_fields__:
    _sc_params = dataclasses.replace(_sc_params, needs_layout_passes=False)


def _f32(*shape):
    return jax.ShapeDtypeStruct(shape, jnp.float32)


@functools.partial(
    pl.kernel, mesh=_mesh, out_type=_f32(NT, NR),
    compiler_params=_sc_params,
    scratch_types=[pltpu.VMEM((CH, K), jnp.int32),
                   pltpu.VMEM((NR,), jnp.float32)])
def _sc_degree(dst_hbm, out_hbm, dst_iv, acc_v):
    c = lax.axis_index("c")
    s = lax.axis_index("s")
    wid = s * NC + c
    pltpu.sync_copy(dst_hbm.at[wid], dst_iv)

    @pl.loop(0, NR, step=16)
    def _(i):
        acc_v[pl.ds(i, 16)] = jnp.zeros((16,), jnp.float32)

    ones = jnp.ones((16,), jnp.float32)

    @pl.loop(0, CH)
    def _(j):
        for k in range(K // 16):
            idx = dst_iv[j, pl.ds(k * 16, 16)]
            plsc.addupdate_scatter(acc_v, [idx], ones)

    pltpu.sync_copy(acc_v, out_hbm.at[wid])


@functools.partial(
    pl.kernel, mesh=_mesh, out_type=_f32(NC, NR, D),
    scratch_types=[pltpu.VMEM((CH, K), jnp.int32),
                   pltpu.VMEM((CH, K), jnp.int32),
                   pltpu.VMEM((K, D), jnp.float32),
                   pltpu.VMEM_SHARED((NR, D), jnp.float32)])
def _sc_aggregate(hs_hbm, src_hbm, dst_hbm, out_hbm,
                  src_iv, dst_iv, rows_v, acc_sh):
    c = lax.axis_index("c")
    s = lax.axis_index("s")
    wid = s * NC + c
    pltpu.sync_copy(src_hbm.at[wid], src_iv)
    pltpu.sync_copy(dst_hbm.at[wid], dst_iv)
    r0 = s * RPS
    pltpu.sync_copy(hs_hbm.at[pl.ds(r0, RPS)], acc_sh.at[pl.ds(r0, RPS)])
    plsc.subcore_barrier()

    @pl.loop(0, CH)
    def _(j):
        pltpu.sync_copy(hs_hbm.at[src_iv.at[j]], rows_v)
        pltpu.sync_copy(rows_v, acc_sh.at[dst_iv.at[j]], add=True)

    plsc.subcore_barrier()
    pltpu.sync_copy(acc_sh.at[pl.ds(r0, RPS)], out_hbm.at[c, pl.ds(r0, RPS)])


@functools.partial(
    pl.kernel, mesh=_mesh, out_type=_f32(NT, NR),
    compiler_params=_sc_params,
    scratch_types=[pltpu.VMEM((CH, K), jnp.int32),
                   pltpu.VMEM((CH, K), jnp.int32),
                   pltpu.VMEM((NR,), jnp.float32),
                   pltpu.VMEM((NR,), jnp.float32)])
def _sc_scalar_aggregate(zs_hbm, src_hbm, dst_hbm, out_hbm,
                         src_iv, dst_iv, zs_v, acc_v):
    c = lax.axis_index("c")
    s = lax.axis_index("s")
    wid = s * NC + c
    pltpu.sync_copy(src_hbm.at[wid], src_iv)
    pltpu.sync_copy(dst_hbm.at[wid], dst_iv)
    pltpu.sync_copy(zs_hbm, zs_v)

    @pl.loop(0, NR, step=16)
    def _(i):
        acc_v[pl.ds(i, 16)] = jnp.zeros((16,), jnp.float32)

    @pl.loop(0, CH)
    def _(j):
        for k in range(K // 16):
            si = src_iv[j, pl.ds(k * 16, 16)]
            di = dst_iv[j, pl.ds(k * 16, 16)]
            vals = plsc.load_gather(zs_v, [si])
            plsc.addupdate_scatter(acc_v, [di], vals)

    pltpu.sync_copy(acc_v, out_hbm.at[wid])


_BLK = 1280
_GRID = NR // _BLK


def _mm_body(x_ref, w_ref, o_ref):
    o_ref[...] = lax.dot(x_ref[...], w_ref[...],
                         precision=lax.Precision.HIGHEST,
                         preferred_element_type=jnp.float32)


def _tc_matmul(xp, W1):
    return pl.pallas_call(
        _mm_body,
        grid=(_GRID,),
        in_specs=[pl.BlockSpec((_BLK, D), lambda i: (i, 0)),
                  pl.BlockSpec((D, D), lambda i: (0, 0))],
        out_specs=pl.BlockSpec((_BLK, D), lambda i: (i, 0)),
        out_shape=_f32(NR, D),
    )(xp, W1)


def _scale_body(hist_ref, h_ref, hs_ref, dinv_ref):
    deg = 1.0 + jnp.sum(hist_ref[...], axis=0)
    dinv = lax.rsqrt(deg)[:, None]
    dinv_ref[...] = dinv
    hs_ref[...] = h_ref[...] * dinv


def _tc_scale(hist, H):
    return pl.pallas_call(
        _scale_body,
        grid=(_GRID,),
        in_specs=[pl.BlockSpec((NT, _BLK), lambda i: (0, i)),
                  pl.BlockSpec((_BLK, D), lambda i: (i, 0))],
        out_specs=[pl.BlockSpec((_BLK, D), lambda i: (i, 0)),
                   pl.BlockSpec((_BLK, 1), lambda i: (i, 0))],
        out_shape=[_f32(NR, D), _f32(NR, 1)],
    )(hist, H)


def _mid_body(p_ref, hs_ref, dinv_ref, b1_ref, w2_ref, zs_ref):
    d = dinv_ref[...]
    h = d * (p_ref[0] + p_ref[1] - hs_ref[...]) + b1_ref[...]
    h = jnp.maximum(h, 0.0)
    z = jnp.sum(h * w2_ref[...], axis=1, keepdims=True)
    zs_ref[...] = d * z


def _tc_mid(P, Hs, dinv, b1r, w2r):
    return pl.pallas_call(
        _mid_body,
        grid=(_GRID,),
        in_specs=[pl.BlockSpec((NC, _BLK, D), lambda i: (0, i, 0)),
                  pl.BlockSpec((_BLK, D), lambda i: (i, 0)),
                  pl.BlockSpec((_BLK, 1), lambda i: (i, 0)),
                  pl.BlockSpec((1, D), lambda i: (0, 0)),
                  pl.BlockSpec((1, D), lambda i: (0, 0))],
        out_specs=pl.BlockSpec((_BLK, 1), lambda i: (i, 0)),
        out_shape=_f32(NR, 1),
    )(P, Hs, dinv, b1r, w2r)


def _final_body(q_ref, zs_ref, dinv_ref, b2_ref, o_ref):
    q = jnp.sum(q_ref[...], axis=0)[:, None]
    o_ref[...] = dinv_ref[...] * (q + zs_ref[...]) + b2_ref[...]


def _tc_final(Q, zs, dinv, b2r):
    return pl.pallas_call(
        _final_body,
        grid=(_GRID,),
        in_specs=[pl.BlockSpec((NT, _BLK), lambda i: (0, i)),
                  pl.BlockSpec((_BLK, 1), lambda i: (i, 0)),
                  pl.BlockSpec((_BLK, 1), lambda i: (i, 0)),
                  pl.BlockSpec((1, 1), lambda i: (0, 0))],
        out_specs=pl.BlockSpec((_BLK, 1), lambda i: (i, 0)),
        out_shape=_f32(NR, 1),
    )(Q, zs, dinv, b2r)


def kernel(x, edge_index, W1, b1, W2, b2):
    xp = jnp.pad(x, ((0, NR - N), (0, 0)))
    pad = jnp.full((EPAD - E,), NR - 1, dtype=edge_index.dtype)
    src = jnp.concatenate([edge_index[0], pad]).reshape(NT, CH, K)
    dst = jnp.concatenate([edge_index[1], pad]).reshape(NT, CH, K)

    hist = _sc_degree(dst)
    H = _tc_matmul(xp, W1)
    Hs, dinv = _tc_scale(hist, H)
    P = _sc_aggregate(Hs, src, dst)
    zs = _tc_mid(P, Hs, dinv, b1.reshape(1, D), W2.reshape(1, D))
    Q = _sc_scalar_aggregate(zs.reshape(NR), src, dst)
    return _tc_final(Q, zs, dinv, b2.reshape(1, 1))[:N]


# speedup vs baseline: 1.9759x; 1.9759x over previous
import dataclasses
import functools

import jax
import jax.numpy as jnp
from jax import lax
from jax.experimental import pallas as pl
from jax.experimental.pallas import tpu as pltpu
from jax.experimental.pallas import tpu_sc as plsc

N = 10000
E = 320000
D = 128
NR = 10240
NC, NS = 2, 16
NT = NC * NS
K = 128
CH = 79
EPT = K * CH
EPAD = NT * EPT
RPS = NR // NS

_mesh = plsc.VectorSubcoreMesh(core_axis_name="c", subcore_axis_name="s")

_sc_params = pltpu.CompilerParams()
if "needs_layout_passes" in pltpu.CompilerParams.__dataclass_fields__:
    _sc_params = dataclasses.replace(_sc_params, needs_layout_passes=False)


def _f32(*shape):
    return jax.ShapeDtypeStruct(shape, jnp.float32)


@functools.partial(
    pl.kernel, mesh=_mesh, out_type=_f32(NT, NR),
    compiler_params=_sc_params,
    scratch_types=[pltpu.VMEM((CH, K), jnp.int32),
                   pltpu.VMEM((NR,), jnp.float32)])
def _sc_degree(dst_hbm, out_hbm, dst_iv, acc_v):
    c = lax.axis_index("c")
    s = lax.axis_index("s")
    wid = s * NC + c
    pltpu.sync_copy(dst_hbm.at[wid], dst_iv)

    @pl.loop(0, NR, step=16)
    def _(i):
        acc_v[pl.ds(i, 16)] = jnp.zeros((16,), jnp.float32)

    ones = jnp.ones((16,), jnp.float32)

    @pl.loop(0, CH)
    def _(j):
        for k in range(K // 16):
            idx = dst_iv[j, pl.ds(k * 16, 16)]
            plsc.addupdate_scatter(acc_v, [idx], ones)

    pltpu.sync_copy(acc_v, out_hbm.at[wid])


@functools.partial(
    pl.kernel, mesh=_mesh, out_type=_f32(NC, NR, D),
    scratch_types=[pltpu.VMEM((CH, K), jnp.int32),
                   pltpu.VMEM((CH, K), jnp.int32),
                   pltpu.VMEM((K, D), jnp.float32),
                   pltpu.VMEM_SHARED((NR, D), jnp.float32)])
def _sc_aggregate(hs_hbm, src_hbm, dst_hbm, out_hbm,
                  src_iv, dst_iv, rows_v, acc_sh):
    c = lax.axis_index("c")
    s = lax.axis_index("s")
    wid = s * NC + c
    pltpu.sync_copy(src_hbm.at[wid], src_iv)
    pltpu.sync_copy(dst_hbm.at[wid], dst_iv)
    r0 = s * RPS
    pltpu.sync_copy(hs_hbm.at[pl.ds(r0, RPS)], acc_sh.at[pl.ds(r0, RPS)])
    plsc.subcore_barrier()

    @pl.loop(0, CH)
    def _(j):
        pltpu.sync_copy(hs_hbm.at[src_iv.at[j]], rows_v)
        pltpu.sync_copy(rows_v, acc_sh.at[dst_iv.at[j]], add=True)

    plsc.subcore_barrier()
    pltpu.sync_copy(acc_sh.at[pl.ds(r0, RPS)], out_hbm.at[c, pl.ds(r0, RPS)])


@functools.partial(
    pl.kernel, mesh=_mesh, out_type=_f32(NT, NR),
    compiler_params=_sc_params,
    scratch_types=[pltpu.VMEM((CH, K), jnp.int32),
                   pltpu.VMEM((CH, K), jnp.int32),
                   pltpu.VMEM((NR,), jnp.float32),
                   pltpu.VMEM((NR,), jnp.float32)])
def _sc_scalar_aggregate(zs_hbm, src_hbm, dst_hbm, out_hbm,
                         src_iv, dst_iv, zs_v, acc_v):
    c = lax.axis_index("c")
    s = lax.axis_index("s")
    wid = s * NC + c
    pltpu.sync_copy(src_hbm.at[wid], src_iv)
    pltpu.sync_copy(dst_hbm.at[wid], dst_iv)
    pltpu.sync_copy(zs_hbm, zs_v)

    @pl.loop(0, NR, step=16)
    def _(i):
        acc_v[pl.ds(i, 16)] = jnp.zeros((16,), jnp.float32)

    @pl.loop(0, CH)
    def _(j):
        for k in range(K // 16):
            si = src_iv[j, pl.ds(k * 16, 16)]
            di = dst_iv[j, pl.ds(k * 16, 16)]
            vals = plsc.load_gather(zs_v, [si])
            plsc.addupdate_scatter(acc_v, [di], vals)

    pltpu.sync_copy(acc_v, out_hbm.at[wid])


_BLK = 1280
_GRID = NR // _BLK


def _mm_body(x_ref, w_ref, o_ref):
    o_ref[...] = lax.dot(x_ref[...], w_ref[...],
                         precision=lax.Precision.HIGHEST,
                         preferred_element_type=jnp.float32)


def _tc_matmul(xp, W1):
    return pl.pallas_call(
        _mm_body,
        grid=(_GRID,),
        in_specs=[pl.BlockSpec((_BLK, D), lambda i: (i, 0)),
                  pl.BlockSpec((D, D), lambda i: (0, 0))],
        out_specs=pl.BlockSpec((_BLK, D), lambda i: (i, 0)),
        out_shape=_f32(NR, D),
    )(xp, W1)


def _scale_body(hist_ref, h_ref, hs_ref, dinv_ref):
    deg = 1.0 + jnp.sum(hist_ref[...], axis=0)
    dinv = lax.rsqrt(deg)[:, None]
    dinv_ref[...] = dinv
    hs_ref[...] = h_ref[...] * dinv


def _tc_scale(hist, H):
    return pl.pallas_call(
        _scale_body,
        grid=(_GRID,),
        in_specs=[pl.BlockSpec((NT, _BLK), lambda i: (0, i)),
                  pl.BlockSpec((_BLK, D), lambda i: (i, 0))],
        out_specs=[pl.BlockSpec((_BLK, D), lambda i: (i, 0)),
                   pl.BlockSpec((_BLK, 1), lambda i: (i, 0))],
        out_shape=[_f32(NR, D), _f32(NR, 1)],
    )(hist, H)


def _mid_body(p_ref, hs_ref, dinv_ref, b1_ref, w2_ref, zs_ref):
    d = dinv_ref[...]
    h = d * (p_ref[0] + p_ref[1] - hs_ref[...]) + b1_ref[...]
    h = jnp.maximum(h, 0.0)
    z = jnp.sum(h * w2_ref[...], axis=1, keepdims=True)
    zs_ref[...] = d * z


def _tc_mid(P, Hs, dinv, b1r, w2r):
    return pl.pallas_call(
        _mid_body,
        grid=(_GRID,),
        in_specs=[pl.BlockSpec((NC, _BLK, D), lambda i: (0, i, 0)),
                  pl.BlockSpec((_BLK, D), lambda i: (i, 0)),
                  pl.BlockSpec((_BLK, 1), lambda i: (i, 0)),
                  pl.BlockSpec((1, D), lambda i: (0, 0)),
                  pl.BlockSpec((1, D), lambda i: (0, 0))],
        out_specs=pl.BlockSpec((_BLK, 1), lambda i: (i, 0)),
        out_shape=_f32(NR, 1),
    )(P, Hs, dinv, b1r, w2r)


def _final_body(q_ref, zs_ref, dinv_ref, b2_ref, o_ref):
    q = jnp.sum(q_ref[...], axis=0)[:, None]
    o_ref[...] = dinv_ref[...] * (q + zs_ref[...]) + b2_ref[...]


def _tc_final(Q, zs, dinv, b2r):
    return pl.pallas_call(
        _final_body,
        grid=(_GRID,),
        in_specs=[pl.BlockSpec((NT, _BLK), lambda i: (0, i)),
                  pl.BlockSpec((_BLK, 1), lambda i: (i, 0)),
                  pl.BlockSpec((_BLK, 1), lambda i: (i, 0)),
                  pl.BlockSpec((1, 1), lambda i: (0, 0))],
        out_specs=pl.BlockSpec((_BLK, 1), lambda i: (i, 0)),
        out_shape=_f32(NR, 1),
    )(Q, zs, dinv, b2r)


def kernel(x, edge_index, W1, b1, W2, b2):
    xp = jnp.pad(x, ((0, NR - N), (0, 0)))
    pad = N + jnp.arange(EPAD - E, dtype=edge_index.dtype) % (NR - N)
    src = jnp.concatenate([edge_index[0], pad]).reshape(NT, CH, K)
    dst = jnp.concatenate([edge_index[1], pad]).reshape(NT, CH, K)

    hist = _sc_degree(dst)
    H = _tc_matmul(xp, W1)
    Hs, dinv = _tc_scale(hist, H)
    P = _sc_aggregate(Hs, src, dst)
    zs = _tc_mid(P, Hs, dinv, b1.reshape(1, D), W2.reshape(1, D))
    Q = _sc_scalar_aggregate(zs.reshape(NR), src, dst)
    return _tc_final(Q, zs, dinv, b2.reshape(1, 1))[:N]


# speedup vs baseline: 2.3945x; 1.2119x over previous
import dataclasses
import functools

import jax
import jax.numpy as jnp
from jax import lax
from jax.experimental import pallas as pl
from jax.experimental.pallas import tpu as pltpu
from jax.experimental.pallas import tpu_sc as plsc

N = 10000
E = 320000
D = 128
NR = 10240
NC, NS = 2, 16
NT = NC * NS
K = 128
CH = 80
RING = 4
EPT = K * CH
EPAD = NT * EPT
RPS = NR // NS

_mesh = plsc.VectorSubcoreMesh(core_axis_name="c", subcore_axis_name="s")

_sc_params = pltpu.CompilerParams()
if "needs_layout_passes" in pltpu.CompilerParams.__dataclass_fields__:
    _sc_params = dataclasses.replace(_sc_params, needs_layout_passes=False)


def _f32(*shape):
    return jax.ShapeDtypeStruct(shape, jnp.float32)


@functools.partial(
    pl.kernel, mesh=_mesh, out_type=_f32(NT, NR),
    compiler_params=_sc_params,
    scratch_types=[pltpu.VMEM((CH, K), jnp.int32),
                   pltpu.VMEM((NR,), jnp.float32)])
def _sc_degree(dst_hbm, out_hbm, dst_iv, acc_v):
    c = lax.axis_index("c")
    s = lax.axis_index("s")
    wid = s * NC + c
    pltpu.sync_copy(dst_hbm.at[wid], dst_iv)

    @pl.loop(0, NR, step=16)
    def _(i):
        acc_v[pl.ds(i, 16)] = jnp.zeros((16,), jnp.float32)

    ones = jnp.ones((16,), jnp.float32)

    @pl.loop(0, CH)
    def _(j):
        for k in range(K // 16):
            idx = dst_iv[j, pl.ds(k * 16, 16)]
            plsc.addupdate_scatter(acc_v, [idx], ones)

    pltpu.sync_copy(acc_v, out_hbm.at[wid])


@functools.partial(
    pl.kernel, mesh=_mesh, out_type=_f32(NC, NR, D),
    scratch_types=[pltpu.VMEM((RING, K), jnp.int32),
                   pltpu.VMEM((RING, K), jnp.int32),
                   pltpu.VMEM((K, D), jnp.float32),
                   pltpu.VMEM((K, D), jnp.float32),
                   pltpu.VMEM_SHARED((NR, D), jnp.float32)]
                  + [pltpu.SemaphoreType.DMA] * (2 + 2 * RING))
def _sc_aggregate(hs_hbm, src_hbm, dst_hbm, out_hbm,
                  src_ring, dst_ring, rows0, rows1, acc_sh, *sems):
    rows = (rows0, rows1)
    gsems = sems[:2]
    ssems = sems[2:2 + RING]
    dsems = sems[2 + RING:]
    c = lax.axis_index("c")
    s = lax.axis_index("s")
    wid = s * NC + c
    for m in range(RING):
        pltpu.async_copy(src_hbm.at[wid, m], src_ring.at[m], ssems[m])
        pltpu.async_copy(dst_hbm.at[wid, m], dst_ring.at[m], dsems[m])
    pltpu.make_async_copy(src_hbm.at[wid, 0], src_ring.at[0], ssems[0]).wait()
    pltpu.async_copy(hs_hbm.at[src_ring.at[0]], rows0, gsems[0])
    r0 = s * RPS
    pltpu.sync_copy(hs_hbm.at[pl.ds(r0, RPS)], acc_sh.at[pl.ds(r0, RPS)])
    plsc.subcore_barrier()

    @pl.loop(0, CH, step=RING)
    def _(j):
        for t in range(RING):
            buf, gsem = rows[t % 2], gsems[t % 2]
            nbuf, ngsem = rows[(t + 1) % 2], gsems[(t + 1) % 2]
            ns = (t + 1) % RING
            pltpu.make_async_copy(hs_hbm.at[src_ring.at[t]], buf, gsem).wait()

            @pl.when(j + RING < CH)
            def _():
                pltpu.async_copy(src_hbm.at[wid, j + RING + t],
                                 src_ring.at[t], ssems[t])

            if t < RING - 1:
                pltpu.make_async_copy(src_hbm.at[wid, 0],
                                      src_ring.at[ns], ssems[ns]).wait()
                pltpu.async_copy(hs_hbm.at[src_ring.at[ns]], nbuf, ngsem)
            else:
                @pl.when(j + RING < CH)
                def _():
                    pltpu.make_async_copy(src_hbm.at[wid, 0],
                                          src_ring.at[ns], ssems[ns]).wait()
                    pltpu.async_copy(hs_hbm.at[src_ring.at[ns]], nbuf, ngsem)

            pltpu.make_async_copy(dst_hbm.at[wid, 0],
                                  dst_ring.at[t], dsems[t]).wait()
            pltpu.sync_copy(buf, acc_sh.at[dst_ring.at[t]], add=True)

            @pl.when(j + RING < CH)
            def _():
                pltpu.async_copy(dst_hbm.at[wid, j + RING + t],
                                 dst_ring.at[t], dsems[t])

    plsc.subcore_barrier()
    pltpu.sync_copy(acc_sh.at[pl.ds(r0, RPS)], out_hbm.at[c, pl.ds(r0, RPS)])


@functools.partial(
    pl.kernel, mesh=_mesh, out_type=_f32(NT, NR),
    compiler_params=_sc_params,
    scratch_types=[pltpu.VMEM((CH, K), jnp.int32),
                   pltpu.VMEM((CH, K), jnp.int32),
                   pltpu.VMEM((NR,), jnp.float32),
                   pltpu.VMEM((NR,), jnp.float32)])
def _sc_scalar_aggregate(zs_hbm, src_hbm, dst_hbm, out_hbm,
                         src_iv, dst_iv, zs_v, acc_v):
    c = lax.axis_index("c")
    s = lax.axis_index("s")
    wid = s * NC + c
    pltpu.sync_copy(src_hbm.at[wid], src_iv)
    pltpu.sync_copy(dst_hbm.at[wid], dst_iv)
    pltpu.sync_copy(zs_hbm, zs_v)

    @pl.loop(0, NR, step=16)
    def _(i):
        acc_v[pl.ds(i, 16)] = jnp.zeros((16,), jnp.float32)

    @pl.loop(0, CH)
    def _(j):
        for k in range(K // 16):
            si = src_iv[j, pl.ds(k * 16, 16)]
            di = dst_iv[j, pl.ds(k * 16, 16)]
            vals = plsc.load_gather(zs_v, [si])
            plsc.addupdate_scatter(acc_v, [di], vals)

    pltpu.sync_copy(acc_v, out_hbm.at[wid])


_BLK = 1280
_GRID = NR // _BLK


def _mm_body(x_ref, w_ref, o_ref):
    o_ref[...] = lax.dot(x_ref[...], w_ref[...],
                         precision=lax.Precision.HIGHEST,
                         preferred_element_type=jnp.float32)


def _tc_matmul(xp, W1):
    return pl.pallas_call(
        _mm_body,
        grid=(_GRID,),
        in_specs=[pl.BlockSpec((_BLK, D), lambda i: (i, 0)),
                  pl.BlockSpec((D, D), lambda i: (0, 0))],
        out_specs=pl.BlockSpec((_BLK, D), lambda i: (i, 0)),
        out_shape=_f32(NR, D),
    )(xp, W1)


def _scale_body(hist_ref, h_ref, hs_ref, dinv_ref):
    deg = 1.0 + jnp.sum(hist_ref[...], axis=0)
    dinv = lax.rsqrt(deg)[:, None]
    dinv_ref[...] = dinv
    hs_ref[...] = h_ref[...] * dinv


def _tc_scale(hist, H):
    return pl.pallas_call(
        _scale_body,
        grid=(_GRID,),
        in_specs=[pl.BlockSpec((NT, _BLK), lambda i: (0, i)),
                  pl.BlockSpec((_BLK, D), lambda i: (i, 0))],
        out_specs=[pl.BlockSpec((_BLK, D), lambda i: (i, 0)),
                   pl.BlockSpec((_BLK, 1), lambda i: (i, 0))],
        out_shape=[_f32(NR, D), _f32(NR, 1)],
    )(hist, H)


def _mid_body(p_ref, hs_ref, dinv_ref, b1_ref, w2_ref, zs_ref):
    d = dinv_ref[...]
    h = d * (p_ref[0] + p_ref[1] - hs_ref[...]) + b1_ref[...]
    h = jnp.maximum(h, 0.0)
    z = jnp.sum(h * w2_ref[...], axis=1, keepdims=True)
    zs_ref[...] = d * z


def _tc_mid(P, Hs, dinv, b1r, w2r):
    return pl.pallas_call(
        _mid_body,
        grid=(_GRID,),
        in_specs=[pl.BlockSpec((NC, _BLK, D), lambda i: (0, i, 0)),
                  pl.BlockSpec((_BLK, D), lambda i: (i, 0)),
                  pl.BlockSpec((_BLK, 1), lambda i: (i, 0)),
                  pl.BlockSpec((1, D), lambda i: (0, 0)),
                  pl.BlockSpec((1, D), lambda i: (0, 0))],
        out_specs=pl.BlockSpec((_BLK, 1), lambda i: (i, 0)),
        out_shape=_f32(NR, 1),
    )(P, Hs, dinv, b1r, w2r)


def _final_body(q_ref, zs_ref, dinv_ref, b2_ref, o_ref):
    q = jnp.sum(q_ref[...], axis=0)[:, None]
    o_ref[...] = dinv_ref[...] * (q + zs_ref[...]) + b2_ref[...]


def _tc_final(Q, zs, dinv, b2r):
    return pl.pallas_call(
        _final_body,
        grid=(_GRID,),
        in_specs=[pl.BlockSpec((NT, _BLK), lambda i: (0, i)),
                  pl.BlockSpec((_BLK, 1), lambda i: (i, 0)),
                  pl.BlockSpec((_BLK, 1), lambda i: (i, 0)),
                  pl.BlockSpec((1, 1), lambda i: (0, 0))],
        out_specs=pl.BlockSpec((_BLK, 1), lambda i: (i, 0)),
        out_shape=_f32(NR, 1),
    )(Q, zs, dinv, b2r)


def kernel(x, edge_index, W1, b1, W2, b2):
    xp = jnp.pad(x, ((0, NR - N), (0, 0)))
    pad = N + jnp.arange(EPAD - E, dtype=edge_index.dtype) % (NR - N)
    src = jnp.concatenate([edge_index[0], pad]).reshape(NT, CH, K)
    dst = jnp.concatenate([edge_index[1], pad]).reshape(NT, CH, K)

    hist = _sc_degree(dst)
    H = _tc_matmul(xp, W1)
    Hs, dinv = _tc_scale(hist, H)
    P = _sc_aggregate(Hs, src, dst)
    zs = _tc_mid(P, Hs, dinv, b1.reshape(1, D), W2.reshape(1, D))
    Q = _sc_scalar_aggregate(zs.reshape(NR), src, dst)
    return _tc_final(Q, zs, dinv, b2.reshape(1, 1))[:N]
